# Initial kernel scaffold; baseline (speedup 1.0000x reference)
import jax
import jax.numpy as jnp
from jax.experimental import pallas as pl


def kernel(x, edge_index, W1, b1, W2, b2):
    raise NotImplementedError("write your pallas kernel here")


# speedup vs baseline: 12.0818x; 12.0818x over previous
import functools

import jax
import jax.numpy as jnp
from jax import lax
from jax.experimental import pallas as pl
from jax.experimental.pallas import tpu as pltpu
from jax.experimental.pallas import tpu_sc as plsc

N = 10000
E = 320000
D = 128
H = 128
C = 40
CP = 128

NCORES = 2
NSUB = 16
NW = NCORES * NSUB
CHUNK = 128
CPW = -(-E // (NW * CHUNK))
EP = NW * CPW * CHUNK
NACC = 10240
RPT = NACC // NSUB
TRASH = N

_mesh = plsc.VectorSubcoreMesh(core_axis_name="c", subcore_axis_name="s")


def _make_scatter(F, gather):
    @functools.partial(
        pl.kernel,
        out_type=jax.ShapeDtypeStruct((NCORES, NACC, F), jnp.float32),
        mesh=_mesh,
        scratch_types=[
            pltpu.VMEM((CPW, CHUNK), jnp.int32),
            pltpu.VMEM((CPW, CHUNK), jnp.int32),
            pltpu.VMEM((CHUNK, F), jnp.float32),
            pltpu.VMEM_SHARED((NACC, F), jnp.float32),
            pltpu.SemaphoreType.DMA,
        ],
    )
    def scat(table_hbm, srcw_hbm, dstw_hbm, zeros_hbm, out_hbm,
             idx_s, idx_d, rows, acc, sem):
        c = lax.axis_index("c")
        s = lax.axis_index("s")
        w = s * NCORES + c
        r0 = s * RPT
        pltpu.sync_copy(zeros_hbm.at[pl.ds(r0, RPT)], acc.at[pl.ds(r0, RPT)])
        pltpu.sync_copy(srcw_hbm.at[w], idx_s)
        pltpu.sync_copy(dstw_hbm.at[w], idx_d)
        if not gather:
            pltpu.sync_copy(table_hbm.at[pl.ds(0, CHUNK)], rows)
        plsc.subcore_barrier()

        def body(j, carry):
            if gather:
                pltpu.async_copy(table_hbm.at[idx_s.at[j]], rows, sem).wait()
            pltpu.sync_copy(rows, acc.at[idx_d.at[j]], add=True)
            return carry

        lax.fori_loop(0, CPW, body, 0)
        plsc.subcore_barrier()
        pltpu.sync_copy(acc.at[pl.ds(r0, RPT)],
                        out_hbm.at[c, pl.ds(r0, RPT)])

    return scat


_scat = _make_scatter(H, gather=True)
_deg_kernel = _make_scatter(H, gather=False)

_BN = 1000


def _dis_block(deg_ref):
    d = deg_ref[0, :, 0:1] + deg_ref[1, :, 0:1] + 1.0
    return lax.rsqrt(d)


def _tc1_body(x_ref, w_ref, deg_ref, y_ref):
    dis = _dis_block(deg_ref)
    y_ref[...] = jnp.dot(x_ref[...], w_ref[...],
                         preferred_element_type=jnp.float32) * dis


def _tc2_body(z_ref, y1_ref, deg_ref, b1_ref, w2_ref, y2_ref):
    dis = _dis_block(deg_ref)
    h = (z_ref[0] + z_ref[1] + y1_ref[...]) * dis + b1_ref[...]
    h = jnp.maximum(h, 0.0)
    y2_ref[...] = jnp.dot(h, w2_ref[...],
                          preferred_element_type=jnp.float32) * dis


def _tc3_body(z_ref, y2_ref, deg_ref, b2_ref, out_ref):
    dis = _dis_block(deg_ref)
    v = (z_ref[0] + z_ref[1] + y2_ref[...]) * dis + b2_ref[...]
    col = lax.broadcasted_iota(jnp.int32, v.shape, 1)
    valid = col < C
    m = jnp.max(jnp.where(valid, v, -1e30), axis=1, keepdims=True)
    e = jnp.where(valid, jnp.exp(v - m), 0.0)
    lse = jnp.log(jnp.sum(e, axis=1, keepdims=True)) + m
    out_ref[...] = v - lse


def _deg_spec():
    return pl.BlockSpec((NCORES, _BN, H), lambda i: (0, i, 0))


def kernel(x, edge_index, W1, b1, W2, b2):
    src = edge_index[0].astype(jnp.int32)
    dst = edge_index[1].astype(jnp.int32)
    pad = EP - E
    srcw = jnp.concatenate([src, jnp.zeros((pad,), jnp.int32)]
                           ).reshape(NW, CPW, CHUNK)
    dstw = jnp.concatenate([dst, jnp.full((pad,), TRASH, jnp.int32)]
                           ).reshape(NW, CPW, CHUNK)

    zeros128 = jnp.zeros((NACC, H), jnp.float32)
    ones128 = jnp.ones((CHUNK, H), jnp.float32)
    degp = _deg_kernel(ones128, srcw, dstw, zeros128)

    y1 = pl.pallas_call(
        _tc1_body,
        grid=(N // _BN,),
        in_specs=[pl.BlockSpec((_BN, D), lambda i: (i, 0)),
                  pl.BlockSpec((D, H), lambda i: (0, 0)),
                  _deg_spec()],
        out_specs=pl.BlockSpec((_BN, H), lambda i: (i, 0)),
        out_shape=jax.ShapeDtypeStruct((N, H), jnp.float32),
    )(x, W1, degp)

    z1p = _scat(y1, srcw, dstw, zeros128)

    W2p = jnp.pad(W2, ((0, 0), (0, CP - C)))
    y2 = pl.pallas_call(
        _tc2_body,
        grid=(N // _BN,),
        in_specs=[pl.BlockSpec((NCORES, _BN, H), lambda i: (0, i, 0)),
                  pl.BlockSpec((_BN, H), lambda i: (i, 0)),
                  _deg_spec(),
                  pl.BlockSpec((1, H), lambda i: (0, 0)),
                  pl.BlockSpec((H, CP), lambda i: (0, 0))],
        out_specs=pl.BlockSpec((_BN, CP), lambda i: (i, 0)),
        out_shape=jax.ShapeDtypeStruct((N, CP), jnp.float32),
    )(z1p, y1, degp, b1.reshape(1, H), W2p)

    z2p = _scat(y2, srcw, dstw, zeros128)

    b2p = jnp.pad(b2, (0, CP - C)).reshape(1, CP)
    out = pl.pallas_call(
        _tc3_body,
        grid=(N // _BN,),
        in_specs=[pl.BlockSpec((NCORES, _BN, CP), lambda i: (0, i, 0)),
                  pl.BlockSpec((_BN, CP), lambda i: (i, 0)),
                  _deg_spec(),
                  pl.BlockSpec((1, CP), lambda i: (0, 0))],
        out_specs=pl.BlockSpec((_BN, CP), lambda i: (i, 0)),
        out_shape=jax.ShapeDtypeStruct((N, CP), jnp.float32),
    )(z2p, y2, degp, b2p)

    return out[:, :C]
